# baseline (speedup 1.0000x reference)
import functools

import jax
import jax.numpy as jnp
from jax import lax
from jax.experimental import pallas as pl
from jax.experimental.pallas import tpu as pltpu
from jax.experimental.pallas import tpu_sc as plsc

VOCAB = 1000000
EMBED = 64
BATCH = 16384
P = 4
N = 20
ROWS = P + N

NUM_WORKERS = 32
PER_W = BATCH // NUM_WORKERS
CHUNK = 32
NCHUNK = PER_W // CHUNK
CTX = CHUNK * ROWS
GSLICE = 128


def _sc_dots(input_labels, pos_labels, neg_labels, center, back):
    mesh = plsc.VectorSubcoreMesh(
        core_axis_name="c", subcore_axis_name="s", num_cores=2, num_subcores=16)

    @functools.partial(
        pl.kernel,
        mesh=mesh,
        out_type=jax.ShapeDtypeStruct((BATCH, 32), jnp.float32),
        compiler_params=pltpu.CompilerParams(
            needs_layout_passes=False, use_tc_tiling_on_sc=False),
        scratch_types=[
            pltpu.VMEM((CHUNK,), jnp.int32),
            pltpu.VMEM((CHUNK, P), jnp.int32),
            pltpu.VMEM((CHUNK, N), jnp.int32),
            pltpu.VMEM((CTX,), jnp.int32),
            pltpu.VMEM((CHUNK, EMBED), jnp.float32),
            pltpu.VMEM((CTX, EMBED), jnp.float32),
            pltpu.VMEM((CHUNK, 32), jnp.float32),
            pltpu.SemaphoreType.DMA,
            pltpu.SemaphoreType.DMA,
        ],
    )
    def k(inlab_hbm, pos_hbm, neg_hbm, center_hbm, back_hbm, out_hbm,
          idxc_v, idxp_v, idxn_v, idxf_v, in_rows, ctx_rows, dots_v,
          sem_c, sem_b):
        wid = lax.axis_index("s") * 2 + lax.axis_index("c")
        lane = lax.iota(jnp.int32, 16)

        @pl.loop(0, NCHUNK)
        def _chunk(kk):
            base = wid * PER_W + kk * CHUNK
            pltpu.sync_copy(inlab_hbm.at[pl.ds(base, CHUNK)], idxc_v)
            pltpu.sync_copy(pos_hbm.at[pl.ds(base, CHUNK)], idxp_v)
            pltpu.sync_copy(neg_hbm.at[pl.ds(base, CHUNK)], idxn_v)
            cdesc = pltpu.async_copy(center_hbm.at[idxc_v], in_rows, sem_c)
            for g in range(CHUNK * P // 16):
                fl = 16 * g + lane
                idxf_v[pl.ds(16 * g, 16)] = plsc.load_gather(
                    idxp_v, [fl >> 2, fl & 3])
            for i in range(CHUNK):
                o = CHUNK * P + N * i
                idxf_v[pl.ds(o, 16)] = idxn_v[i, pl.ds(0, 16)]
                idxf_v[pl.ds(o + N - 16, 16)] = idxn_v[i, pl.ds(N - 16, 16)]
            gds = []
            for j in range(CTX // GSLICE):
                gds.append(pltpu.async_copy(
                    back_hbm.at[idxf_v.at[pl.ds(j * GSLICE, GSLICE)]],
                    ctx_rows.at[pl.ds(j * GSLICE, GSLICE)],
                    sem_b))
            cdesc.wait()
            for d in gds:
                d.wait()

            @pl.loop(0, CHUNK)
            def _elem(b):
                ins = [in_rows[b, pl.ds(16 * q, 16)] for q in range(4)]
                lo = jnp.zeros((16,), jnp.float32)
                hi = jnp.zeros((16,), jnp.float32)
                for r in range(ROWS):
                    if r < P:
                        ro = P * b + r
                    else:
                        ro = CHUNK * P + N * b + (r - P)
                    acc = ins[0] * ctx_rows[ro, pl.ds(0, 16)]
                    for q in range(1, 4):
                        acc = acc + ins[q] * ctx_rows[ro, pl.ds(16 * q, 16)]
                    d = jnp.sum(acc)
                    if r < 16:
                        lo = jnp.where(lane == r, d, lo)
                    else:
                        hi = jnp.where(lane == (r - 16), d, hi)
                dots_v[b, pl.ds(0, 16)] = lo
                dots_v[b, pl.ds(16, 16)] = hi

            pltpu.sync_copy(dots_v, out_hbm.at[pl.ds(base, CHUNK)])

    return k(input_labels, pos_labels, neg_labels, center, back)


def _logsig(x):
    return jnp.minimum(x, 0.0) - jnp.log1p(jnp.exp(-jnp.abs(x)))


def _loss_body(d_ref, o_ref):
    x = d_ref[...]
    pos = x[:, 0:P]
    neg = x[:, P:ROWS]
    lp = jnp.sum(_logsig(pos), axis=1)
    ln = jnp.sum(_logsig(-neg), axis=1)
    o_ref[...] = -(lp + ln)


def _tc_loss(dots):
    blk = 1024
    return pl.pallas_call(
        _loss_body,
        grid=(BATCH // blk,),
        in_specs=[pl.BlockSpec((blk, 32), lambda i: (i, 0))],
        out_specs=pl.BlockSpec((blk,), lambda i: (i,)),
        out_shape=jax.ShapeDtypeStruct((BATCH,), jnp.float32),
    )(dots)


def kernel(input_labels, pos_labels, neg_labels, center_embedding, back_embedding):
    dots = _sc_dots(input_labels.astype(jnp.int32),
                    pos_labels.astype(jnp.int32),
                    neg_labels.astype(jnp.int32),
                    center_embedding, back_embedding)
    return _tc_loss(dots)


# speedup vs baseline: 1.0115x; 1.0115x over previous
import functools

import jax
import jax.numpy as jnp
from jax import lax
from jax.experimental import pallas as pl
from jax.experimental.pallas import tpu as pltpu
from jax.experimental.pallas import tpu_sc as plsc

VOCAB = 1000000
EMBED = 64
BATCH = 16384
P = 4
N = 20
ROWS = P + N

NUM_WORKERS = 32
PER_W = BATCH // NUM_WORKERS
CHUNK = 32
NCHUNK = PER_W // CHUNK
CTX = CHUNK * ROWS
GSLICE = 128


def _sc_dots(input_labels, pos_labels, neg_labels, center, back):
    mesh = plsc.VectorSubcoreMesh(
        core_axis_name="c", subcore_axis_name="s", num_cores=2, num_subcores=16)

    @functools.partial(
        pl.kernel,
        mesh=mesh,
        out_type=jax.ShapeDtypeStruct((BATCH, 32), jnp.float32),
        compiler_params=pltpu.CompilerParams(
            needs_layout_passes=False, use_tc_tiling_on_sc=False),
        scratch_types=[
            pltpu.VMEM((CHUNK,), jnp.int32),
            pltpu.VMEM((P, CHUNK), jnp.int32),
            pltpu.VMEM((N, CHUNK), jnp.int32),
            pltpu.VMEM((CTX,), jnp.int32),
            pltpu.VMEM((CHUNK, EMBED), jnp.float32),
            pltpu.VMEM((CTX, EMBED), jnp.float32),
            pltpu.VMEM((CHUNK, 32), jnp.float32),
            pltpu.SemaphoreType.DMA,
            pltpu.SemaphoreType.DMA,
        ],
    )
    def k(inlab_hbm, pos_hbm, neg_hbm, center_hbm, back_hbm, out_hbm,
          idxc_v, idxp_v, idxn_v, idxf_v, in_rows, ctx_rows, dots_v,
          sem_c, sem_b):
        wid = lax.axis_index("s") * 2 + lax.axis_index("c")
        lane = lax.iota(jnp.int32, 16)

        @pl.loop(0, NCHUNK)
        def _chunk(kk):
            base = wid * PER_W + kk * CHUNK
            pltpu.sync_copy(inlab_hbm.at[pl.ds(base, CHUNK)], idxc_v)
            pltpu.sync_copy(pos_hbm.at[:, pl.ds(base, CHUNK)], idxp_v)
            pltpu.sync_copy(neg_hbm.at[:, pl.ds(base, CHUNK)], idxn_v)
            cdesc = pltpu.async_copy(center_hbm.at[idxc_v], in_rows, sem_c)
            for j in range(P):
                for h in range(CHUNK // 16):
                    idxf_v[pl.ds(j * CHUNK + 16 * h, 16)] = (
                        idxp_v[j, pl.ds(16 * h, 16)])
            for j in range(N):
                for h in range(CHUNK // 16):
                    idxf_v[pl.ds(CHUNK * P + j * CHUNK + 16 * h, 16)] = (
                        idxn_v[j, pl.ds(16 * h, 16)])
            gds = []
            for j in range(CTX // GSLICE):
                gds.append(pltpu.async_copy(
                    back_hbm.at[idxf_v.at[pl.ds(j * GSLICE, GSLICE)]],
                    ctx_rows.at[pl.ds(j * GSLICE, GSLICE)],
                    sem_b))
            cdesc.wait()
            for d in gds:
                d.wait()

            @pl.loop(0, CHUNK)
            def _elem(b):
                ins = [in_rows[b, pl.ds(16 * q, 16)] for q in range(4)]
                lo = jnp.zeros((16,), jnp.float32)
                hi = jnp.zeros((16,), jnp.float32)
                for r in range(ROWS):
                    if r < P:
                        ro = r * CHUNK + b
                    else:
                        ro = CHUNK * P + (r - P) * CHUNK + b
                    acc = ins[0] * ctx_rows[ro, pl.ds(0, 16)]
                    for q in range(1, 4):
                        acc = acc + ins[q] * ctx_rows[ro, pl.ds(16 * q, 16)]
                    d = jnp.sum(acc)
                    if r < 16:
                        lo = jnp.where(lane == r, d, lo)
                    else:
                        hi = jnp.where(lane == (r - 16), d, hi)
                dots_v[b, pl.ds(0, 16)] = lo
                dots_v[b, pl.ds(16, 16)] = hi

            pltpu.sync_copy(dots_v, out_hbm.at[pl.ds(base, CHUNK)])

    return k(input_labels, pos_labels, neg_labels, center, back)


def _logsig(x):
    return jnp.minimum(x, 0.0) - jnp.log1p(jnp.exp(-jnp.abs(x)))


def _loss_body(d_ref, o_ref):
    x = d_ref[...]
    pos = x[:, 0:P]
    neg = x[:, P:ROWS]
    lp = jnp.sum(_logsig(pos), axis=1)
    ln = jnp.sum(_logsig(-neg), axis=1)
    o_ref[...] = -(lp + ln)


def _tc_loss(dots):
    blk = 1024
    return pl.pallas_call(
        _loss_body,
        grid=(BATCH // blk,),
        in_specs=[pl.BlockSpec((blk, 32), lambda i: (i, 0))],
        out_specs=pl.BlockSpec((blk,), lambda i: (i,)),
        out_shape=jax.ShapeDtypeStruct((BATCH,), jnp.float32),
    )(dots)


def kernel(input_labels, pos_labels, neg_labels, center_embedding, back_embedding):
    dots = _sc_dots(input_labels.astype(jnp.int32),
                    pos_labels.astype(jnp.int32).T,
                    neg_labels.astype(jnp.int32).T,
                    center_embedding, back_embedding)
    return _tc_loss(dots)


# speedup vs baseline: 1.2442x; 1.2300x over previous
import functools

import jax
import jax.numpy as jnp
from jax import lax
from jax.experimental import pallas as pl
from jax.experimental.pallas import tpu as pltpu
from jax.experimental.pallas import tpu_sc as plsc

VOCAB = 1000000
EMBED = 64
BATCH = 16384
P = 4
N = 20
ROWS = P + N

NUM_WORKERS = 32
PER_W = BATCH // NUM_WORKERS
CHUNK = 32
NCHUNK = PER_W // CHUNK
CTX = CHUNK * ROWS
GSLICE = 128

ACHUNK = 128
NACHUNK = PER_W // ACHUNK

_MESH = dict(core_axis_name="c", subcore_axis_name="s",
             num_cores=2, num_subcores=16)


def _sc_center_rows(input_labels, center):
    @functools.partial(
        pl.kernel,
        mesh=plsc.VectorSubcoreMesh(**_MESH),
        out_type=jax.ShapeDtypeStruct((BATCH * EMBED,), jnp.float32),
        compiler_params=pltpu.CompilerParams(
            needs_layout_passes=False, use_tc_tiling_on_sc=True),
        scratch_types=[
            pltpu.VMEM((ACHUNK,), jnp.int32),
            pltpu.VMEM((ACHUNK * EMBED,), jnp.float32),
            pltpu.VMEM((16, 8, EMBED), jnp.float32),
            pltpu.SemaphoreType.DMA,
        ],
    )
    def ka(inlab_hbm, center_hbm, out_hbm, lab_v, rows_v, tiles_v, sem):
        wid = lax.axis_index("s") * 2 + lax.axis_index("c")
        lane = lax.iota(jnp.int32, 16)

        @pl.loop(0, NACHUNK)
        def _chunk(kk):
            base = wid * PER_W + kk * ACHUNK
            pltpu.sync_copy(inlab_hbm.at[pl.ds(base, ACHUNK)], lab_v)
            for g in range(ACHUNK // 16):
                labs = lab_v[pl.ds(16 * g, 16)]
                vs = []
                descs = []
                for s in range(16):
                    v = jnp.sum(jnp.where(lane == s, labs, 0))
                    vs.append(v)
                    descs.append(pltpu.async_copy(
                        center_hbm.at[pl.ds(8 * (v >> 3), 8)],
                        tiles_v.at[s], sem))
                for d in descs:
                    d.wait()
                for s in range(16):
                    r = vs[s] & 7
                    o = (16 * g + s) * EMBED
                    for q in range(4):
                        rows_v[pl.ds(o + 16 * q, 16)] = (
                            tiles_v[s, r, pl.ds(16 * q, 16)])
            pltpu.sync_copy(rows_v,
                            out_hbm.at[pl.ds(base * EMBED, ACHUNK * EMBED)])

    return ka(input_labels, center)


def _sc_dots(inrows_flat, pos_labels, neg_labels, back):
    @functools.partial(
        pl.kernel,
        mesh=plsc.VectorSubcoreMesh(**_MESH),
        out_type=jax.ShapeDtypeStruct((BATCH, 32), jnp.float32),
        compiler_params=pltpu.CompilerParams(
            needs_layout_passes=False, use_tc_tiling_on_sc=False),
        scratch_types=[
            pltpu.VMEM((CHUNK * EMBED,), jnp.float32),
            pltpu.VMEM((P, CHUNK), jnp.int32),
            pltpu.VMEM((N, CHUNK), jnp.int32),
            pltpu.VMEM((CTX,), jnp.int32),
            pltpu.VMEM((CTX, EMBED), jnp.float32),
            pltpu.VMEM((CHUNK, 32), jnp.float32),
            pltpu.SemaphoreType.DMA,
            pltpu.SemaphoreType.DMA,
        ],
    )
    def kb(inrows_hbm, pos_hbm, neg_hbm, back_hbm, out_hbm,
           in_v, idxp_v, idxn_v, idxf_v, ctx_rows, dots_v, sem_c, sem_b):
        wid = lax.axis_index("s") * 2 + lax.axis_index("c")
        lane = lax.iota(jnp.int32, 16)

        @pl.loop(0, NCHUNK)
        def _chunk(kk):
            base = wid * PER_W + kk * CHUNK
            cdesc = pltpu.async_copy(
                inrows_hbm.at[pl.ds(base * EMBED, CHUNK * EMBED)], in_v, sem_c)
            pltpu.sync_copy(pos_hbm.at[:, pl.ds(base, CHUNK)], idxp_v)
            pltpu.sync_copy(neg_hbm.at[:, pl.ds(base, CHUNK)], idxn_v)
            for j in range(P):
                for h in range(CHUNK // 16):
                    idxf_v[pl.ds(j * CHUNK + 16 * h, 16)] = (
                        idxp_v[j, pl.ds(16 * h, 16)])
            for j in range(N):
                for h in range(CHUNK // 16):
                    idxf_v[pl.ds(CHUNK * P + j * CHUNK + 16 * h, 16)] = (
                        idxn_v[j, pl.ds(16 * h, 16)])
            gds = []
            for j in range(CTX // GSLICE):
                gds.append(pltpu.async_copy(
                    back_hbm.at[idxf_v.at[pl.ds(j * GSLICE, GSLICE)]],
                    ctx_rows.at[pl.ds(j * GSLICE, GSLICE)],
                    sem_b))
            cdesc.wait()
            for d in gds:
                d.wait()

            @pl.loop(0, CHUNK)
            def _elem(b):
                ins = [in_v[pl.ds(EMBED * b + 16 * q, 16)] for q in range(4)]
                lo = jnp.zeros((16,), jnp.float32)
                hi = jnp.zeros((16,), jnp.float32)
                for r in range(ROWS):
                    if r < P:
                        ro = r * CHUNK + b
                    else:
                        ro = CHUNK * P + (r - P) * CHUNK + b
                    acc = ins[0] * ctx_rows[ro, pl.ds(0, 16)]
                    for q in range(1, 4):
                        acc = acc + ins[q] * ctx_rows[ro, pl.ds(16 * q, 16)]
                    d = jnp.sum(acc)
                    if r < 16:
                        lo = jnp.where(lane == r, d, lo)
                    else:
                        hi = jnp.where(lane == (r - 16), d, hi)
                dots_v[b, pl.ds(0, 16)] = lo
                dots_v[b, pl.ds(16, 16)] = hi

            pltpu.sync_copy(dots_v, out_hbm.at[pl.ds(base, CHUNK)])

    return kb(inrows_flat, pos_labels, neg_labels, back)


def _logsig(x):
    return jnp.minimum(x, 0.0) - jnp.log1p(jnp.exp(-jnp.abs(x)))


def _loss_body(d_ref, o_ref):
    x = d_ref[...]
    pos = x[:, 0:P]
    neg = x[:, P:ROWS]
    lp = jnp.sum(_logsig(pos), axis=1)
    ln = jnp.sum(_logsig(-neg), axis=1)
    o_ref[...] = -(lp + ln)


def _tc_loss(dots):
    blk = 1024
    return pl.pallas_call(
        _loss_body,
        grid=(BATCH // blk,),
        in_specs=[pl.BlockSpec((blk, 32), lambda i: (i, 0))],
        out_specs=pl.BlockSpec((blk,), lambda i: (i,)),
        out_shape=jax.ShapeDtypeStruct((BATCH,), jnp.float32),
    )(dots)


def kernel(input_labels, pos_labels, neg_labels, center_embedding, back_embedding):
    inrows = _sc_center_rows(input_labels.astype(jnp.int32), center_embedding)
    dots = _sc_dots(inrows,
                    pos_labels.astype(jnp.int32).T,
                    neg_labels.astype(jnp.int32).T,
                    back_embedding)
    return _tc_loss(dots)
